# baseline (device time: 9080 ns/iter reference)
import jax
import jax.numpy as jnp
from jax import lax
from jax.experimental import pallas as pl
from jax.experimental.pallas import tpu as pltpu

N_DEV = 8
K = 8


def kernel(x):
    m, n = x.shape
    B = m // K

    def body(x_hbm, out_hbm, in_buf, out_buf, halo_ref, row0_buf,
             in_sems, out_sems, row0_sem, send_sems, recv_sems):
        my_pos = lax.axis_index("i")
        left = (my_pos - 1) % N_DEV
        right = (my_pos + 1) % N_DEV

        def start_in(k):
            d = pltpu.make_async_copy(
                x_hbm.at[pl.ds(k * B, B)], in_buf.at[k % 2], in_sems.at[k % 2]
            )
            d.start()
            return d

        in_dma = [None] * K
        in_dma[0] = start_in(0)
        in_dma[1] = start_in(1)

        barrier_sem = pltpu.get_barrier_semaphore()
        for nbr in (left, right):
            pl.semaphore_signal(
                barrier_sem, inc=1,
                device_id=(nbr,), device_id_type=pl.DeviceIdType.MESH,
            )
        pl.semaphore_wait(barrier_sem, 2)

        to_left = pltpu.make_async_remote_copy(
            src_ref=x_hbm.at[pl.ds(0, 1)],
            dst_ref=halo_ref.at[1],
            send_sem=send_sems.at[1],
            recv_sem=recv_sems.at[1],
            device_id=(left,),
            device_id_type=pl.DeviceIdType.MESH,
        )
        to_right = pltpu.make_async_remote_copy(
            src_ref=x_hbm.at[pl.ds(m - 1, 1)],
            dst_ref=halo_ref.at[0],
            send_sem=send_sems.at[0],
            recv_sem=recv_sems.at[0],
            device_id=(right,),
            device_id_type=pl.DeviceIdType.MESH,
        )
        to_left.start()
        to_right.start()

        out_dma = [None] * K
        prev_last = None
        x0 = x1 = None

        for k in range(K):
            if k == 0:
                in_dma[0].wait()
            if k + 1 < K:
                in_dma[k + 1].wait()
            if k >= 2:
                out_dma[k - 2].wait()

            chunk = in_buf[k % 2]
            if k == 0:
                x0 = chunk[0:1]
                x1 = chunk[1:2]
                prev_last = chunk[0:1]
            if k + 1 < K:
                next_first = in_buf[(k + 1) % 2][0:1]
            else:
                next_first = chunk[B - 1 : B]

            up = jnp.concatenate([prev_last, chunk[: B - 1]], axis=0)
            down = jnp.concatenate([chunk[1:], next_first], axis=0)
            out_buf[k % 2] = 0.25 * up + 0.5 * chunk + 0.25 * down

            if k == K - 1:
                to_left.wait_recv()
                xm2 = chunk[B - 2 : B - 1]
                xm1 = chunk[B - 1 : B]
                out_buf[k % 2, B - 1 : B] = jnp.where(
                    my_pos == N_DEV - 1,
                    xm1,
                    0.25 * xm2 + 0.5 * xm1 + 0.25 * halo_ref[1],
                )

            prev_last = chunk[B - 1 : B]
            out_dma[k] = pltpu.make_async_copy(
                out_buf.at[k % 2], out_hbm.at[pl.ds(k * B, B)], out_sems.at[k % 2]
            )
            out_dma[k].start()
            if k + 2 < K:
                in_dma[k + 2] = start_in(k + 2)

        out_dma[K - 2].wait()
        out_dma[K - 1].wait()

        to_right.wait_recv()
        row0_buf[:, :] = jnp.where(
            my_pos == 0, x0, 0.25 * halo_ref[0] + 0.5 * x0 + 0.25 * x1
        )
        row0_dma = pltpu.make_async_copy(
            row0_buf, out_hbm.at[pl.ds(0, 1)], row0_sem
        )
        row0_dma.start()
        row0_dma.wait()

        to_left.wait_send()
        to_right.wait_send()

    return pl.pallas_call(
        body,
        out_shape=jax.ShapeDtypeStruct((m, n), x.dtype),
        in_specs=[pl.BlockSpec(memory_space=pl.ANY)],
        out_specs=pl.BlockSpec(memory_space=pl.ANY),
        scratch_shapes=[
            pltpu.VMEM((2, B, n), x.dtype),
            pltpu.VMEM((2, B, n), x.dtype),
            pltpu.VMEM((2, 1, n), x.dtype),
            pltpu.VMEM((1, n), x.dtype),
            pltpu.SemaphoreType.DMA((2,)),
            pltpu.SemaphoreType.DMA((2,)),
            pltpu.SemaphoreType.DMA,
            pltpu.SemaphoreType.DMA((2,)),
            pltpu.SemaphoreType.DMA((2,)),
        ],
        compiler_params=pltpu.CompilerParams(collective_id=0),
    )(x)
